# baseline (speedup 1.0000x reference)
import jax
import jax.numpy as jnp
from jax.experimental import pallas as pl

_K = 20
_PREC = jax.lax.Precision.HIGHEST


def _bf(x):
    return x.astype(jnp.bfloat16)


def _knn_idx(h):
    hb = _bf(h)
    inner = -2.0 * jnp.einsum('bcn,bcm->bnm', hb, hb, precision=_PREC,
                              preferred_element_type=jnp.float32)
    xx = jnp.sum(h ** 2, axis=1)
    dist = -xx[:, :, None] - inner - xx[:, None, :]
    return jax.lax.top_k(dist, _K)[1]


def _graph_feature(x):
    B, C, N = x.shape
    idx = _knn_idx(x)
    x_t = jnp.transpose(x, (0, 2, 1))
    feat = jax.vmap(lambda xt, i: xt[i])(x_t, idx)
    x_rep = jnp.broadcast_to(x_t[:, :, None, :], (B, N, _K, C))
    out = jnp.concatenate([feat - x_rep, x_rep], axis=3)
    return jnp.transpose(out, (0, 3, 1, 2))


def _conv_bn_lrelu(x, W, gamma, beta, act=True):
    y = jnp.einsum('oc,bcnk->bonk', _bf(W), _bf(x), precision=_PREC,
                   preferred_element_type=jnp.float32)
    mean = jnp.mean(y, axis=(0, 2, 3), keepdims=True)
    var = jnp.var(y, axis=(0, 2, 3), keepdims=True)
    yn = (y - mean) / jnp.sqrt(var + 1e-5)
    yn = yn * gamma[None, :, None, None] + beta[None, :, None, None]
    if act:
        yn = jnp.where(yn > 0, yn, 0.2 * yn)
    return yn


def _lrelu_pallas(x):
    def body(x_ref, o_ref):
        v = x_ref[...]
        o_ref[...] = jnp.where(v > 0, v, 0.2 * v)
    return pl.pallas_call(
        body, out_shape=jax.ShapeDtypeStruct(x.shape, x.dtype))(x)


def kernel(x, W1, g1, b1, W2, g2, b2, W3, g3, b3, W4, g4, b4):
    h = _graph_feature(x)
    h = _conv_bn_lrelu(h, W1, g1, b1)
    h = jnp.max(h, axis=-1)
    h = _graph_feature(h)
    h = _conv_bn_lrelu(h, W2, g2, b2)
    h = jnp.max(h, axis=-1)
    h = _graph_feature(h)
    h = _conv_bn_lrelu(h, W3, g3, b3)
    h = jnp.max(h, axis=-1)
    h = _graph_feature(h)
    h = _conv_bn_lrelu(h, W4, g4, b4, act=False)
    h = jnp.max(h, axis=-1)
    return _lrelu_pallas(h)


# speedup vs baseline: 6.8873x; 6.8873x over previous
import functools

import jax
import jax.numpy as jnp
from jax import lax
from jax.experimental import pallas as pl
from jax.experimental.pallas import tpu as pltpu
from jax.experimental.pallas import tpu_sc as plsc

_K = 20
_N = 2048
_B = 8
_TN = 256
_TC = 512
_INTERPRET = False


def _topk_body(ht_ref, o_ref, work_ref):
    b = pl.program_id(0)
    i = pl.program_id(1)
    ht = ht_ref[0]
    xx_full = jnp.sum(ht * ht, axis=0)
    a = ht_ref[0, :, pl.ds(i * _TN, _TN)]
    xx_tile = jnp.sum(a * a, axis=0)
    mm = lax.dot_general(a, ht, (((0,), (0,)), ((), ())),
                         preferred_element_type=jnp.float32)
    inner = -2.0 * mm
    t = (-xx_tile)[:, None] - inner
    dist = t - xx_full[None, :]
    work_ref[...] = dist
    lane = lax.broadcasted_iota(jnp.int32, (_TN, _N), 1)
    base = b * _N
    for j in range(_K):
        w = work_ref[...]
        m = jnp.max(w, axis=1)
        eq = w == m[:, None]
        idx = jnp.min(jnp.where(eq, lane, _N), axis=1)
        o_ref[j, :] = idx + base
        work_ref[...] = jnp.where(lane == idx[:, None], -jnp.inf, w)


def _knn_pallas(ht):
    B, C, N = ht.shape
    return pl.pallas_call(
        _topk_body,
        grid=(B, N // _TN),
        in_specs=[pl.BlockSpec((1, C, N), lambda b, i: (b, 0, 0))],
        out_specs=pl.BlockSpec((_K, _TN), lambda b, i: (0, b * (_N // _TN) + i)),
        out_shape=jax.ShapeDtypeStruct((_K, B * N), jnp.int32),
        scratch_shapes=[pltpu.VMEM((_TN, _N), jnp.float32)],
        interpret=_INTERPRET,
    )(ht)


def _sc_gather(table, gidx):
    R_tot = gidx.shape[0]
    C = table.shape[1]
    NW = 32
    per_w = R_tot // NW
    R = 128
    iters = per_w // R

    @functools.partial(
        pl.kernel,
        mesh=plsc.VectorSubcoreMesh(core_axis_name="c", subcore_axis_name="s"),
        compiler_params=pltpu.CompilerParams(use_tc_tiling_on_sc=False),
        out_type=jax.ShapeDtypeStruct((R_tot, C), jnp.float32),
        scratch_types=[
            pltpu.VMEM((R,), jnp.int32),
            pltpu.VMEM((R, C), jnp.float32),
            pltpu.SemaphoreType.DMA,
        ],
    )
    def k(table_hbm, gidx_hbm, out_hbm, idx_v, rows_v, sem):
        wid = lax.axis_index("s") * 2 + lax.axis_index("c")
        base0 = wid * per_w

        def body(it, carry):
            base = base0 + it * R
            pltpu.sync_copy(gidx_hbm.at[pl.ds(base, R)], idx_v)
            pltpu.async_copy(table_hbm.at[idx_v], rows_v, sem).wait()
            pltpu.sync_copy(rows_v, out_hbm.at[pl.ds(base, R)])
            return carry

        lax.fori_loop(0, iters, body, 0)

    return k(table, gidx)


def _conv_body(g_ref, x_ref, w_ref, z_ref, y_ref):
    nt = pl.program_id(0)
    k = pl.program_id(1)
    x = x_ref[...]
    d = g_ref[...] - x
    cat = jnp.concatenate([d, x], axis=1)
    y = lax.dot_general(cat, w_ref[...], (((1,), (0,)), ((), ())),
                        preferred_element_type=jnp.float32)

    y_ref[...] = y

    @pl.when(k == 0)
    def _first():
        z_ref[...] = y

    @pl.when(k > 0)
    def _rest():
        z_ref[...] = jnp.maximum(z_ref[...], y)


def _conv_max_pallas(G, X, WT):
    BN, C = X.shape
    O = WT.shape[1]
    NT = BN // _TC
    return pl.pallas_call(
        _conv_body,
        grid=(NT, _K),
        in_specs=[
            pl.BlockSpec((_TC, C), lambda nt, k: (k * NT + nt, 0)),
            pl.BlockSpec((_TC, C), lambda nt, k: (nt, 0)),
            pl.BlockSpec((2 * C, O), lambda nt, k: (0, 0)),
        ],
        out_specs=[
            pl.BlockSpec((_TC, O), lambda nt, k: (nt, 0)),
            pl.BlockSpec((_TC, O), lambda nt, k: (k * NT + nt, 0)),
        ],
        out_shape=[
            jax.ShapeDtypeStruct((BN, O), jnp.float32),
            jax.ShapeDtypeStruct((_K * BN, O), jnp.float32),
        ],
        interpret=_INTERPRET,
    )(G, X, WT)


def _bn_body(z_ref, m_ref, v_ref, g_ref, b_ref, o_ref):
    denom = jnp.sqrt(v_ref[...] + 1e-5)
    zn = ((z_ref[...] - m_ref[...]) / denom) * g_ref[...] + b_ref[...]
    o_ref[...] = jnp.where(zn > 0, zn, 0.2 * zn)


def _bn_pallas(Z, mean, var, g, b):
    BN, O = Z.shape
    return pl.pallas_call(
        _bn_body,
        grid=(BN // _TC,),
        in_specs=[
            pl.BlockSpec((_TC, O), lambda nt: (nt, 0)),
            pl.BlockSpec((1, O), lambda nt: (0, 0)),
            pl.BlockSpec((1, O), lambda nt: (0, 0)),
            pl.BlockSpec((1, O), lambda nt: (0, 0)),
            pl.BlockSpec((1, O), lambda nt: (0, 0)),
        ],
        out_specs=pl.BlockSpec((_TC, O), lambda nt: (nt, 0)),
        out_shape=jax.ShapeDtypeStruct((BN, O), jnp.float32),
        interpret=_INTERPRET,
    )(Z, mean.reshape(1, O), var.reshape(1, O),
      g.reshape(1, O), b.reshape(1, O))


def _layer(ht, hp, WT, g, b):
    B, C, N = ht.shape
    O = WT.shape[1]
    gidx = _knn_pallas(ht)
    G = _sc_gather(hp, gidx.reshape(-1))
    Z, Y = _conv_max_pallas(G, hp, WT)
    yy = jnp.transpose(Y.reshape(_K, B, N, O), (1, 3, 2, 0))
    yy = jax.lax.optimization_barrier(yy)
    mean = jnp.mean(yy, axis=(0, 2, 3))
    var = jnp.var(yy, axis=(0, 2, 3))
    h = _bn_pallas(Z, mean, var, g, b)
    ht_next = jnp.transpose(h.reshape(B, N, O), (0, 2, 1))
    return ht_next, h


def kernel(x, W1, g1, b1, W2, g2, b2, W3, g3, b3, W4, g4, b4):
    C0 = x.shape[1]
    ht = jnp.pad(x, ((0, 0), (0, 3), (0, 0)))
    hp = jnp.transpose(ht, (0, 2, 1)).reshape(_B * _N, 8)
    W1T = jnp.concatenate([
        jnp.pad(jnp.transpose(W1[:, :C0]), ((0, 3), (0, 0))),
        jnp.pad(jnp.transpose(W1[:, C0:]), ((0, 3), (0, 0))),
    ], axis=0)
    ht, hp = _layer(ht, hp, W1T, g1, b1)
    for (W, g, b) in ((W2, g2, b2), (W3, g3, b3), (W4, g4, b4)):
        ht, hp = _layer(ht, hp, jnp.transpose(W), g, b)
    return ht
